# baseline (speedup 1.0000x reference)
import functools

import jax
import jax.numpy as jnp
from jax import lax
from jax.experimental import pallas as pl
from jax.experimental.pallas import tpu as pltpu
from jax.experimental.pallas import tpu_sc as plsc

N = 10000
E = 320000
D_NODE = 128
G = 64
N_ACT = 8
EPS = 1e-5

EB = 2000
N_EBLK = E // EB

bf16 = jnp.bfloat16

NC = 2
NS = 16
NW = NC * NS
EW = E // NW
GCH = 200
NCH = EW // GCH
GSUB = 40
GNSUB = GCH // GSUB
SCH = 400
SNCH = EW // SCH
SUB = 80
NSUB = SCH // SUB
NZR = 1000
EBH = E // EB


def _dotf(a, b):
    return jnp.dot(a, b, preferred_element_type=jnp.float32)


def _k0_body(x_ref, w_ref, tad_ref, tb_ref):
    t = _dotf(x_ref[...], w_ref[...])
    tad_ref[...] = t[:, 0:128]
    tb_ref[...] = t[:, 128:192]


def _node_tables(x16, w16):
    return pl.pallas_call(
        _k0_body,
        out_shape=[
            jax.ShapeDtypeStruct((N, 128), jnp.float32),
            jax.ShapeDtypeStruct((N, 64), jnp.float32),
        ],
    )(x16, w16)


def _sc_gather_body(tad, tbp, row, col, gad, gbp,
                    idxr, idxc, bufa, bufb, sema, semb):
    wid = lax.axis_index("s") * NC + lax.axis_index("c")

    def body(i, carry):
        base = wid * EW + i * GCH
        pltpu.sync_copy(row.at[pl.ds(base, GCH)], idxr)
        pltpu.sync_copy(col.at[pl.ds(base, GCH)], idxc)
        cps = []
        for j in range(GNSUB):
            sl = pl.ds(j * GSUB, GSUB)
            cps.append(pltpu.async_copy(tad.at[idxr.at[sl]], bufa.at[sl], sema))
            cps.append(pltpu.async_copy(tbp.at[idxc.at[sl]], bufb.at[sl], semb))
        for cp in cps:
            cp.wait()
        pltpu.sync_copy(bufa, gad.at[pl.ds(base, GCH)])
        pltpu.sync_copy(bufb, gbp.at[pl.ds(base, GCH)])
        return carry

    lax.fori_loop(0, NCH, body, 0)


def _sc_gather(tad, tb, row, col):
    f32 = jnp.float32
    fn = pl.kernel(
        _sc_gather_body,
        out_type=[
            jax.ShapeDtypeStruct((E, 128), f32),
            jax.ShapeDtypeStruct((E, 64), f32),
        ],
        mesh=plsc.VectorSubcoreMesh(core_axis_name="c", subcore_axis_name="s"),
        scratch_types=[
            pltpu.VMEM((GCH,), jnp.int32),
            pltpu.VMEM((GCH,), jnp.int32),
            pltpu.VMEM((GCH, 128), f32),
            pltpu.VMEM((GCH, 64), f32),
            pltpu.SemaphoreType.DMA,
            pltpu.SemaphoreType.DMA,
        ],
        compiler_params=pltpu.CompilerParams(use_tc_tiling_on_sc=False),
    )
    return fn(tad, tb, row, col)


def _sc_scatter_body(m2, col1, out, idxw, mbuf, zbuf, acc):
    c = lax.axis_index("c")
    s = lax.axis_index("s")
    wid = s * NC + c

    def zb(i, carry):
        zbuf[i, :] = jnp.zeros((16,), jnp.float32)
        return carry

    lax.fori_loop(0, NZR, zb, 0)

    @pl.when(s < N // NZR)
    def _():
        pltpu.sync_copy(zbuf, acc.at[pl.ds(s * NZR, NZR)])

    plsc.subcore_barrier()

    def body(i, carry):
        base = wid * EW + i * SCH
        pltpu.sync_copy(m2.at[pl.ds(base, SCH)], mbuf)
        for j in range(NSUB):
            pltpu.sync_copy(col1.at[pl.ds(base + j * SUB, SUB)], idxw)
            pltpu.sync_copy(mbuf.at[pl.ds(j * SUB, SUB)],
                            acc.at[idxw], add=True)
        return carry

    lax.fori_loop(0, SNCH, body, 0)
    plsc.subcore_barrier()

    @pl.when(s < N // NZR)
    def _():
        pltpu.sync_copy(acc.at[pl.ds(s * NZR, NZR)],
                        out.at[c].at[pl.ds(s * NZR, NZR)])


def _sc_scatter(m_aug, col):
    f32 = jnp.float32
    fn = pl.kernel(
        _sc_scatter_body,
        out_type=jax.ShapeDtypeStruct((NC, N, 16), f32),
        mesh=plsc.VectorSubcoreMesh(core_axis_name="c", subcore_axis_name="s"),
        scratch_types=[
            pltpu.VMEM((SUB,), jnp.int32),
            pltpu.VMEM((SCH, 16), f32),
            pltpu.VMEM((NZR, 16), f32),
            pltpu.VMEM_SHARED((N, 16), f32),
        ],
        compiler_params=pltpu.CompilerParams(use_tc_tiling_on_sc=False),
    )
    return fn(m_aug, col)


def _k2a_body(ga_ref, gb_ref, ea_ref, wc_ref, b1_ref, h1_ref, st_ref):
    i = pl.program_id(0)
    h1 = (ga_ref[:, 0:64] + gb_ref[...]
          + _dotf(ea_ref[...].astype(bf16), wc_ref[...])
          + b1_ref[0, :][None, :])
    h1_ref[...] = h1

    @pl.when(i == 0)
    def _():
        st_ref[...] = jnp.zeros_like(st_ref)

    st_ref[0, :] += jnp.sum(h1, axis=0)
    st_ref[1, :] += jnp.sum(h1 * h1, axis=0)


def _edge_pass_a(gA, gB, ea16, wc16, b1):
    return pl.pallas_call(
        _k2a_body,
        grid=(EBH,),
        in_specs=[
            pl.BlockSpec((EB, 128), lambda i: (i, 0)),
            pl.BlockSpec((EB, 64), lambda i: (i, 0)),
            pl.BlockSpec((EB, 16), lambda i: (i, 0)),
            pl.BlockSpec((16, 64), lambda i: (0, 0)),
            pl.BlockSpec((1, 64), lambda i: (0, 0)),
        ],
        out_specs=[
            pl.BlockSpec((EB, 64), lambda i: (i, 0)),
            pl.BlockSpec((8, 64), lambda i: (0, 0)),
        ],
        out_shape=[
            jax.ShapeDtypeStruct((E, 64), jnp.float32),
            jax.ShapeDtypeStruct((8, 64), jnp.float32),
        ],
    )(gA, gB, ea16, wc16, b1)


def _k2b_body(h1_ref, gd_ref, st1_ref, g1_ref, be1_ref, w2_ref, b2_ref,
              w1e_ref, c2_ref, h2_ref, st_ref):
    i = pl.program_id(0)
    mu = st1_ref[0, :] * (1.0 / E)
    var = st1_ref[1, :] * (1.0 / E) - mu * mu
    sd = jnp.sqrt(var + EPS)
    hn = jnp.maximum((h1_ref[...] - mu[None, :]) / sd[None, :]
                     * g1_ref[0, :][None, :] + be1_ref[0, :][None, :], 0.0)
    e_out = _dotf(hn.astype(bf16), w2_ref[...]) + b2_ref[0, :][None, :]
    h2 = (gd_ref[:, 64:128] + _dotf(e_out.astype(bf16), w1e_ref[...])
          + c2_ref[0, :][None, :])
    h2_ref[...] = h2

    @pl.when(i == 0)
    def _():
        st_ref[...] = jnp.zeros_like(st_ref)

    st_ref[0, :] += jnp.sum(h2, axis=0)
    st_ref[1, :] += jnp.sum(h2 * h2, axis=0)


def _edge_pass_b(h1, gD, st1, g1, be1, w2p16, b2p, w1ep16, c2):
    return pl.pallas_call(
        _k2b_body,
        grid=(EBH,),
        in_specs=[
            pl.BlockSpec((EB, 64), lambda i: (i, 0)),
            pl.BlockSpec((EB, 128), lambda i: (i, 0)),
            pl.BlockSpec((8, 64), lambda i: (0, 0)),
            pl.BlockSpec((1, 64), lambda i: (0, 0)),
            pl.BlockSpec((1, 64), lambda i: (0, 0)),
            pl.BlockSpec((64, 16), lambda i: (0, 0)),
            pl.BlockSpec((1, 16), lambda i: (0, 0)),
            pl.BlockSpec((16, 64), lambda i: (0, 0)),
            pl.BlockSpec((1, 64), lambda i: (0, 0)),
        ],
        out_specs=[
            pl.BlockSpec((EB, 64), lambda i: (i, 0)),
            pl.BlockSpec((8, 64), lambda i: (0, 0)),
        ],
        out_shape=[
            jax.ShapeDtypeStruct((E, 64), jnp.float32),
            jax.ShapeDtypeStruct((8, 64), jnp.float32),
        ],
    )(h1, gD, st1, g1, be1, w2p16, b2p, w1ep16, c2)


def _k2c_body(h2_ref, st2_ref, g2_ref, be2_ref, w2_ref, b2_ref, m_ref):
    mu = st2_ref[0, :] * (1.0 / E)
    var = st2_ref[1, :] * (1.0 / E) - mu * mu
    sd = jnp.sqrt(var + EPS)
    hn = jnp.maximum((h2_ref[...] - mu[None, :]) / sd[None, :]
                     * g2_ref[0, :][None, :] + be2_ref[0, :][None, :], 0.0)
    m_ref[...] = _dotf(hn.astype(bf16), w2_ref[...]) + b2_ref[0, :][None, :]


def _edge_pass_c(h2, st2, g2, be2, w2p16, b2p):
    return pl.pallas_call(
        _k2c_body,
        grid=(EBH,),
        in_specs=[
            pl.BlockSpec((EB, 64), lambda i: (i, 0)),
            pl.BlockSpec((8, 64), lambda i: (0, 0)),
            pl.BlockSpec((1, 64), lambda i: (0, 0)),
            pl.BlockSpec((1, 64), lambda i: (0, 0)),
            pl.BlockSpec((64, 16), lambda i: (0, 0)),
            pl.BlockSpec((1, 16), lambda i: (0, 0)),
        ],
        out_specs=pl.BlockSpec((EB, 16), lambda i: (i, 0)),
        out_shape=jax.ShapeDtypeStruct((E, 16), jnp.float32),
    )(h2, st2, g2, be2, w2p16, b2p)


def _k4_body(acc_ref, batch_ref, rw1_ref, rb1_ref, rg1_ref,
             rbe1_ref, rw2_ref, rb2_ref, out_ref):
    acc = acc_ref[0] + acc_ref[1]
    deg = jnp.maximum(acc[:, 10:11], 1.0)
    node16 = acc / deg
    iota = lax.broadcasted_iota(jnp.int32, (N, G), 1)
    onehot = (batch_ref[...] == iota).astype(jnp.float32)
    cnt = jnp.sum(onehot, axis=0)
    u16 = lax.dot_general(onehot, node16, (((0,), (0,)), ((), ())),
                          preferred_element_type=jnp.float32,
                          precision=lax.Precision.HIGHEST)
    u16 = u16 / jnp.maximum(cnt, 1.0)[:, None]
    h = (_dotf(u16.astype(bf16), rw1_ref[...])
         + rb1_ref[0, :][None, :])
    mu = jnp.mean(h, axis=0)
    var = jnp.mean(h * h, axis=0) - mu * mu
    sd = jnp.sqrt(var + EPS)
    hn = jnp.maximum((h - mu[None, :]) / sd[None, :]
                     * rg1_ref[0, :][None, :] + rbe1_ref[0, :][None, :], 0.0)
    out_ref[...] = (_dotf(hn.astype(bf16), rw2_ref[...])
                    + rb2_ref[0, :][None, :])


def _readout(acc, batch2d, rw1p16, rb1, rg1, rbe1, rw216, rb2):
    return pl.pallas_call(
        _k4_body,
        out_shape=jax.ShapeDtypeStruct((G, N_ACT), jnp.float32),
    )(acc, batch2d, rw1p16, rb1, rg1, rbe1, rw216, rb2)


def kernel(x, edge_index, edge_attr, batch,
           eW1, eb1, eg1, ebeta1, eW2, eb2,
           nW1, nb1, ng1, nbeta1, nW2, nb2,
           rW1, rb1, rg1, rbeta1, rW2, rb2):
    f32 = jnp.float32
    row = edge_index[0].astype(jnp.int32)
    col = edge_index[1].astype(jnp.int32)

    w_all16 = jnp.concatenate(
        [eW1[0:128], nW1[0:128], eW1[128:256]],
        axis=1).astype(bf16)
    wc16 = eW1[256:272].astype(bf16)
    b1 = eb1.reshape(1, 64)
    g1 = eg1.reshape(1, 64)
    be1 = ebeta1.reshape(1, 64)
    ew2p16 = jnp.zeros((64, 16), f32).at[:, 0:10].set(eW2).astype(bf16)
    eb2p = jnp.zeros((16,), f32).at[0:10].set(eb2).reshape(1, 16)
    nw1ep16 = jnp.zeros((16, 64), f32).at[0:10, :].set(nW1[128:138]).astype(bf16)
    c2 = nb1.reshape(1, 64)
    g2 = ng1.reshape(1, 64)
    be2 = nbeta1.reshape(1, 64)
    nw2p16 = jnp.zeros((64, 16), f32).at[:, 0:10].set(nW2).astype(bf16)
    nb2p = jnp.zeros((16,), f32).at[0:10].set(nb2).at[10].set(1.0)
    nb2p = nb2p.reshape(1, 16)
    rw1p16 = jnp.zeros((16, 64), f32).at[0:10, :].set(rW1).astype(bf16)
    rw216 = rW2.astype(bf16)

    t_ad, t_b = _node_tables(x.astype(bf16), w_all16)

    gAD, gB = _sc_gather(t_ad, t_b, row, col)

    h1, st1 = _edge_pass_a(gAD, gB, edge_attr, wc16, b1)
    h2, st2 = _edge_pass_b(h1, gAD, st1, g1, be1, ew2p16, eb2p,
                           nw1ep16, c2)
    m_aug = _edge_pass_c(h2, st2, g2, be2, nw2p16, nb2p)

    acc = _sc_scatter(m_aug, col)

    batch2d = batch.astype(jnp.int32).reshape(N, 1)
    return _readout(acc, batch2d, rw1p16, rb1.reshape(1, 64),
                    rg1.reshape(1, 64), rbeta1.reshape(1, 64),
                    rw216, rb2.reshape(1, N_ACT))


# speedup vs baseline: 1.0854x; 1.0854x over previous
import functools

import jax
import jax.numpy as jnp
from jax import lax
from jax.experimental import pallas as pl
from jax.experimental.pallas import tpu as pltpu
from jax.experimental.pallas import tpu_sc as plsc

N = 10000
E = 320000
D_NODE = 128
G = 64
N_ACT = 8
EPS = 1e-5

EB = 2000
N_EBLK = E // EB

bf16 = jnp.bfloat16

NC = 2
NS = 16
NW = NC * NS
EW = E // NW
GCH = 400
NCH = EW // GCH
GSUB = 80
GNSUB = GCH // GSUB
SCH = 400
SNCH = EW // SCH
SUB = 80
NSUB = SCH // SUB
NZR = 1000
EBH = E // EB


def _dotf(a, b):
    return jnp.dot(a, b, preferred_element_type=jnp.float32)


def _k0_body(x_ref, w_ref, t_ref):
    t_ref[...] = _dotf(x_ref[...], w_ref[...])


def _node_tables(x16, w16):
    return pl.pallas_call(
        _k0_body,
        out_shape=jax.ShapeDtypeStruct((N, 256), jnp.float32),
    )(x16, w16)


def _sc_gather_body(tad, tbp, row, col, gad, gbp,
                    idxr, idxc, bufa, bufb, sema, semb):
    wid = lax.axis_index("s") * NC + lax.axis_index("c")

    def body(i, carry):
        base = wid * EW + i * GCH
        pltpu.sync_copy(row.at[pl.ds(base, GCH)], idxr)
        pltpu.sync_copy(col.at[pl.ds(base, GCH)], idxc)
        cps = []
        for j in range(GNSUB):
            sl = pl.ds(j * GSUB, GSUB)
            cps.append(pltpu.async_copy(tad.at[idxr.at[sl]], bufa.at[sl], sema))
            cps.append(pltpu.async_copy(tbp.at[idxc.at[sl]], bufb.at[sl], semb))
        for cp in cps:
            cp.wait()
        pltpu.sync_copy(bufa, gad.at[pl.ds(base, GCH)])
        pltpu.sync_copy(bufb, gbp.at[pl.ds(base, GCH)])
        return carry

    lax.fori_loop(0, NCH, body, 0)


def _sc_gather(tad, tbp, row, col):
    f32 = jnp.float32
    fn = pl.kernel(
        _sc_gather_body,
        out_type=[
            jax.ShapeDtypeStruct((E, 128), f32),
            jax.ShapeDtypeStruct((E, 128), f32),
        ],
        mesh=plsc.VectorSubcoreMesh(core_axis_name="c", subcore_axis_name="s"),
        scratch_types=[
            pltpu.VMEM((GCH,), jnp.int32),
            pltpu.VMEM((GCH,), jnp.int32),
            pltpu.VMEM((GCH, 128), f32),
            pltpu.VMEM((GCH, 128), f32),
            pltpu.SemaphoreType.DMA,
            pltpu.SemaphoreType.DMA,
        ],
    )
    return fn(tad, tbp, row, col)


def _sc_scatter_body(m2, col1, out, idxw, mbuf, zbuf, acc):
    c = lax.axis_index("c")
    s = lax.axis_index("s")
    wid = s * NC + c

    def zb(i, carry):
        zbuf[i, :] = jnp.zeros((16,), jnp.float32)
        return carry

    lax.fori_loop(0, NZR, zb, 0)

    @pl.when(s < N // NZR)
    def _():
        pltpu.sync_copy(zbuf, acc.at[pl.ds(s * NZR, NZR)])

    plsc.subcore_barrier()

    def body(i, carry):
        base = wid * EW + i * SCH
        pltpu.sync_copy(m2.at[pl.ds(base, SCH)], mbuf)
        for j in range(NSUB):
            pltpu.sync_copy(col1.at[pl.ds(base + j * SUB, SUB)], idxw)
            pltpu.sync_copy(mbuf.at[pl.ds(j * SUB, SUB)],
                            acc.at[idxw], add=True)
        return carry

    lax.fori_loop(0, SNCH, body, 0)
    plsc.subcore_barrier()

    @pl.when(s < N // NZR)
    def _():
        pltpu.sync_copy(acc.at[pl.ds(s * NZR, NZR)],
                        out.at[c].at[pl.ds(s * NZR, NZR)])


def _sc_scatter(m_aug, col):
    f32 = jnp.float32
    fn = pl.kernel(
        _sc_scatter_body,
        out_type=jax.ShapeDtypeStruct((NC, N, 16), f32),
        mesh=plsc.VectorSubcoreMesh(core_axis_name="c", subcore_axis_name="s"),
        scratch_types=[
            pltpu.VMEM((SUB,), jnp.int32),
            pltpu.VMEM((SCH, 16), f32),
            pltpu.VMEM((NZR, 16), f32),
            pltpu.VMEM_SHARED((N, 16), f32),
        ],
        compiler_params=pltpu.CompilerParams(use_tc_tiling_on_sc=False),
    )
    return fn(m_aug, col)


def _k2a_body(ga_ref, gb_ref, ea_ref, wc_ref, b1_ref, h1_ref, st_ref):
    i = pl.program_id(0)
    h1 = (ga_ref[:, 0:64] + gb_ref[:, 0:64]
          + _dotf(ea_ref[...], wc_ref[...])
          + b1_ref[0, :][None, :])
    h1_ref[...] = h1

    @pl.when(i == 0)
    def _():
        st_ref[...] = jnp.zeros_like(st_ref)

    st_ref[0, :] += jnp.sum(h1, axis=0)
    st_ref[1, :] += jnp.sum(h1 * h1, axis=0)


def _edge_pass_a(gA, gB, ea16, wc16, b1):
    return pl.pallas_call(
        _k2a_body,
        grid=(EBH,),
        in_specs=[
            pl.BlockSpec((EB, 128), lambda i: (i, 0)),
            pl.BlockSpec((EB, 128), lambda i: (i, 0)),
            pl.BlockSpec((EB, 16), lambda i: (i, 0)),
            pl.BlockSpec((16, 64), lambda i: (0, 0)),
            pl.BlockSpec((1, 64), lambda i: (0, 0)),
        ],
        out_specs=[
            pl.BlockSpec((EB, 64), lambda i: (i, 0)),
            pl.BlockSpec((8, 64), lambda i: (0, 0)),
        ],
        out_shape=[
            jax.ShapeDtypeStruct((E, 64), jnp.float32),
            jax.ShapeDtypeStruct((8, 64), jnp.float32),
        ],
    )(gA, gB, ea16, wc16, b1)


def _k2b_body(h1_ref, gd_ref, st1_ref, g1_ref, be1_ref, w2_ref, b2_ref,
              w1e_ref, c2_ref, h2_ref, st_ref):
    i = pl.program_id(0)
    mu = st1_ref[0, :] * (1.0 / E)
    var = st1_ref[1, :] * (1.0 / E) - mu * mu
    sd = jnp.sqrt(var + EPS)
    hn = jnp.maximum((h1_ref[...] - mu[None, :]) / sd[None, :]
                     * g1_ref[0, :][None, :] + be1_ref[0, :][None, :], 0.0)
    e_out = _dotf(hn.astype(bf16), w2_ref[...]) + b2_ref[0, :][None, :]
    h2 = (gd_ref[:, 64:128] + _dotf(e_out.astype(bf16), w1e_ref[...])
          + c2_ref[0, :][None, :])
    h2_ref[...] = h2

    @pl.when(i == 0)
    def _():
        st_ref[...] = jnp.zeros_like(st_ref)

    st_ref[0, :] += jnp.sum(h2, axis=0)
    st_ref[1, :] += jnp.sum(h2 * h2, axis=0)


def _edge_pass_b(h1, gD, st1, g1, be1, w2p16, b2p, w1ep16, c2):
    return pl.pallas_call(
        _k2b_body,
        grid=(EBH,),
        in_specs=[
            pl.BlockSpec((EB, 64), lambda i: (i, 0)),
            pl.BlockSpec((EB, 128), lambda i: (i, 0)),
            pl.BlockSpec((8, 64), lambda i: (0, 0)),
            pl.BlockSpec((1, 64), lambda i: (0, 0)),
            pl.BlockSpec((1, 64), lambda i: (0, 0)),
            pl.BlockSpec((64, 16), lambda i: (0, 0)),
            pl.BlockSpec((1, 16), lambda i: (0, 0)),
            pl.BlockSpec((16, 64), lambda i: (0, 0)),
            pl.BlockSpec((1, 64), lambda i: (0, 0)),
        ],
        out_specs=[
            pl.BlockSpec((EB, 64), lambda i: (i, 0)),
            pl.BlockSpec((8, 64), lambda i: (0, 0)),
        ],
        out_shape=[
            jax.ShapeDtypeStruct((E, 64), jnp.float32),
            jax.ShapeDtypeStruct((8, 64), jnp.float32),
        ],
    )(h1, gD, st1, g1, be1, w2p16, b2p, w1ep16, c2)


def _k2c_body(h2_ref, st2_ref, g2_ref, be2_ref, w2_ref, b2_ref, m_ref):
    mu = st2_ref[0, :] * (1.0 / E)
    var = st2_ref[1, :] * (1.0 / E) - mu * mu
    sd = jnp.sqrt(var + EPS)
    hn = jnp.maximum((h2_ref[...] - mu[None, :]) / sd[None, :]
                     * g2_ref[0, :][None, :] + be2_ref[0, :][None, :], 0.0)
    m_ref[...] = _dotf(hn.astype(bf16), w2_ref[...]) + b2_ref[0, :][None, :]


def _edge_pass_c(h2, st2, g2, be2, w2p16, b2p):
    return pl.pallas_call(
        _k2c_body,
        grid=(EBH,),
        in_specs=[
            pl.BlockSpec((EB, 64), lambda i: (i, 0)),
            pl.BlockSpec((8, 64), lambda i: (0, 0)),
            pl.BlockSpec((1, 64), lambda i: (0, 0)),
            pl.BlockSpec((1, 64), lambda i: (0, 0)),
            pl.BlockSpec((64, 16), lambda i: (0, 0)),
            pl.BlockSpec((1, 16), lambda i: (0, 0)),
        ],
        out_specs=pl.BlockSpec((EB, 16), lambda i: (i, 0)),
        out_shape=jax.ShapeDtypeStruct((E, 16), jnp.float32),
    )(h2, st2, g2, be2, w2p16, b2p)


def _k4_body(acc_ref, batch_ref, rw1_ref, rb1_ref, rg1_ref,
             rbe1_ref, rw2_ref, rb2_ref, out_ref):
    acc = acc_ref[0] + acc_ref[1]
    deg = jnp.maximum(acc[:, 10:11], 1.0)
    node16 = acc / deg
    iota = lax.broadcasted_iota(jnp.int32, (N, G), 1)
    onehot = (batch_ref[...] == iota).astype(jnp.float32)
    cnt = jnp.sum(onehot, axis=0)
    u16 = lax.dot_general(onehot, node16, (((0,), (0,)), ((), ())),
                          preferred_element_type=jnp.float32,
                          precision=lax.Precision.HIGHEST)
    u16 = u16 / jnp.maximum(cnt, 1.0)[:, None]
    h = (_dotf(u16.astype(bf16), rw1_ref[...])
         + rb1_ref[0, :][None, :])
    mu = jnp.mean(h, axis=0)
    var = jnp.mean(h * h, axis=0) - mu * mu
    sd = jnp.sqrt(var + EPS)
    hn = jnp.maximum((h - mu[None, :]) / sd[None, :]
                     * rg1_ref[0, :][None, :] + rbe1_ref[0, :][None, :], 0.0)
    out_ref[...] = (_dotf(hn.astype(bf16), rw2_ref[...])
                    + rb2_ref[0, :][None, :])


def _readout(acc, batch2d, rw1p16, rb1, rg1, rbe1, rw216, rb2):
    return pl.pallas_call(
        _k4_body,
        out_shape=jax.ShapeDtypeStruct((G, N_ACT), jnp.float32),
    )(acc, batch2d, rw1p16, rb1, rg1, rbe1, rw216, rb2)


def kernel(x, edge_index, edge_attr, batch,
           eW1, eb1, eg1, ebeta1, eW2, eb2,
           nW1, nb1, ng1, nbeta1, nW2, nb2,
           rW1, rb1, rg1, rbeta1, rW2, rb2):
    f32 = jnp.float32
    row = edge_index[0].astype(jnp.int32)
    col = edge_index[1].astype(jnp.int32)

    w_all16 = jnp.concatenate(
        [eW1[0:128], nW1[0:128], eW1[128:256],
         jnp.zeros((128, 64), f32)], axis=1).astype(bf16)
    wc16 = eW1[256:272].astype(bf16)
    b1 = eb1.reshape(1, 64)
    g1 = eg1.reshape(1, 64)
    be1 = ebeta1.reshape(1, 64)
    ew2p16 = jnp.zeros((64, 16), f32).at[:, 0:10].set(eW2).astype(bf16)
    eb2p = jnp.zeros((16,), f32).at[0:10].set(eb2).reshape(1, 16)
    nw1ep16 = jnp.zeros((16, 64), f32).at[0:10, :].set(nW1[128:138]).astype(bf16)
    c2 = nb1.reshape(1, 64)
    g2 = ng1.reshape(1, 64)
    be2 = nbeta1.reshape(1, 64)
    nw2p16 = jnp.zeros((64, 16), f32).at[:, 0:10].set(nW2).astype(bf16)
    nb2p = jnp.zeros((16,), f32).at[0:10].set(nb2).at[10].set(1.0)
    nb2p = nb2p.reshape(1, 16)
    rw1p16 = jnp.zeros((16, 64), f32).at[0:10, :].set(rW1).astype(bf16)
    rw216 = rW2.astype(bf16)

    t = _node_tables(x.astype(bf16), w_all16)
    t_ad = t[:, 0:128]
    t_bp = t[:, 128:256]

    gAD, gBP = _sc_gather(t_ad, t_bp, row, col)

    h1, st1 = _edge_pass_a(gAD, gBP, edge_attr.astype(bf16), wc16, b1)
    h2, st2 = _edge_pass_b(h1, gAD, st1, g1, be1, ew2p16, eb2p,
                           nw1ep16, c2)
    m_aug = _edge_pass_c(h2, st2, g2, be2, nw2p16, nb2p)

    acc = _sc_scatter(m_aug, col)

    batch2d = batch.astype(jnp.int32).reshape(N, 1)
    return _readout(acc, batch2d, rw1p16, rb1.reshape(1, 64),
                    rg1.reshape(1, 64), rbeta1.reshape(1, 64),
                    rw216, rb2.reshape(1, N_ACT))


# speedup vs baseline: 1.2547x; 1.1559x over previous
import functools

import jax
import jax.numpy as jnp
from jax import lax
from jax.experimental import pallas as pl
from jax.experimental.pallas import tpu as pltpu
from jax.experimental.pallas import tpu_sc as plsc

N = 10000
E = 320000
D_NODE = 128
G = 64
N_ACT = 8
EPS = 1e-5

EB = 4000
N_EBLK = E // EB

bf16 = jnp.bfloat16

NC = 2
NS = 16
NW = NC * NS
EW = E // NW
GCH = 400
NCH = EW // GCH
GSUB = 80
GNSUB = GCH // GSUB
SCH = 400
SNCH = EW // SCH
SUB = 80
NSUB = SCH // SUB
NZR = 1000
EBH = E // EB


def _dotf(a, b):
    return jnp.dot(a, b, preferred_element_type=jnp.float32)


def _k0_body(x_ref, w_ref, t_ref):
    t_ref[...] = _dotf(x_ref[...], w_ref[...])


def _node_tables(x16, w16):
    return pl.pallas_call(
        _k0_body,
        out_shape=jax.ShapeDtypeStruct((N, 256), jnp.float32),
    )(x16, w16)


def _sc_gather_body(tad, tbp, row, col, gad, gbp,
                    idxr, idxc, bufa, bufb, sema, semb):
    wid = lax.axis_index("s") * NC + lax.axis_index("c")

    def body(i, carry):
        base = wid * EW + i * GCH
        pltpu.sync_copy(row.at[pl.ds(base, GCH)], idxr)
        pltpu.sync_copy(col.at[pl.ds(base, GCH)], idxc)
        cps = []
        for j in range(GNSUB):
            sl = pl.ds(j * GSUB, GSUB)
            cps.append(pltpu.async_copy(tad.at[idxr.at[sl]], bufa.at[sl], sema))
            cps.append(pltpu.async_copy(tbp.at[idxc.at[sl]], bufb.at[sl], semb))
        for cp in cps:
            cp.wait()
        pltpu.sync_copy(bufa, gad.at[pl.ds(base, GCH)])
        pltpu.sync_copy(bufb, gbp.at[pl.ds(base, GCH)])
        return carry

    lax.fori_loop(0, NCH, body, 0)


def _sc_gather(tad, tbp, row, col):
    f32 = jnp.float32
    fn = pl.kernel(
        _sc_gather_body,
        out_type=[
            jax.ShapeDtypeStruct((E, 128), f32),
            jax.ShapeDtypeStruct((E, 128), f32),
        ],
        mesh=plsc.VectorSubcoreMesh(core_axis_name="c", subcore_axis_name="s"),
        scratch_types=[
            pltpu.VMEM((GCH,), jnp.int32),
            pltpu.VMEM((GCH,), jnp.int32),
            pltpu.VMEM((GCH, 128), f32),
            pltpu.VMEM((GCH, 128), f32),
            pltpu.SemaphoreType.DMA,
            pltpu.SemaphoreType.DMA,
        ],
    )
    return fn(tad, tbp, row, col)


def _sc_scatter_body(m2, col1, out, idxw, mbuf, zbuf, acc):
    c = lax.axis_index("c")
    s = lax.axis_index("s")
    wid = s * NC + c

    def zb(i, carry):
        zbuf[i, :] = jnp.zeros((16,), jnp.float32)
        return carry

    lax.fori_loop(0, NZR, zb, 0)

    @pl.when(s < N // NZR)
    def _():
        pltpu.sync_copy(zbuf, acc.at[pl.ds(s * NZR, NZR)])

    plsc.subcore_barrier()

    def body(i, carry):
        base = wid * EW + i * SCH
        pltpu.sync_copy(m2.at[pl.ds(base, SCH)], mbuf)
        for j in range(NSUB):
            pltpu.sync_copy(col1.at[pl.ds(base + j * SUB, SUB)], idxw)
            pltpu.sync_copy(mbuf.at[pl.ds(j * SUB, SUB)],
                            acc.at[idxw], add=True)
        return carry

    lax.fori_loop(0, SNCH, body, 0)
    plsc.subcore_barrier()

    @pl.when(s < N // NZR)
    def _():
        pltpu.sync_copy(acc.at[pl.ds(s * NZR, NZR)],
                        out.at[c].at[pl.ds(s * NZR, NZR)])


def _sc_scatter(m_aug, col):
    f32 = jnp.float32
    fn = pl.kernel(
        _sc_scatter_body,
        out_type=jax.ShapeDtypeStruct((NC, N, 16), f32),
        mesh=plsc.VectorSubcoreMesh(core_axis_name="c", subcore_axis_name="s"),
        scratch_types=[
            pltpu.VMEM((SUB,), jnp.int32),
            pltpu.VMEM((SCH, 16), f32),
            pltpu.VMEM((NZR, 16), f32),
            pltpu.VMEM_SHARED((N, 16), f32),
        ],
        compiler_params=pltpu.CompilerParams(use_tc_tiling_on_sc=False),
    )
    return fn(m_aug, col)


def _k2a_body(ga_ref, gb_ref, ea_ref, wc_ref, b1_ref, h1_ref, st_ref):
    i = pl.program_id(0)
    h1 = (ga_ref[:, 0:64] + gb_ref[:, 0:64]
          + _dotf(ea_ref[...], wc_ref[...])
          + b1_ref[0, :][None, :])
    h1_ref[...] = h1

    @pl.when(i == 0)
    def _():
        st_ref[...] = jnp.zeros_like(st_ref)

    st_ref[0, :] += jnp.sum(h1, axis=0)
    st_ref[1, :] += jnp.sum(h1 * h1, axis=0)


def _edge_pass_a(gA, gB, ea16, wc16, b1):
    return pl.pallas_call(
        _k2a_body,
        grid=(EBH,),
        in_specs=[
            pl.BlockSpec((EB, 128), lambda i: (i, 0)),
            pl.BlockSpec((EB, 128), lambda i: (i, 0)),
            pl.BlockSpec((EB, 16), lambda i: (i, 0)),
            pl.BlockSpec((16, 64), lambda i: (0, 0)),
            pl.BlockSpec((1, 64), lambda i: (0, 0)),
        ],
        out_specs=[
            pl.BlockSpec((EB, 64), lambda i: (i, 0)),
            pl.BlockSpec((8, 64), lambda i: (0, 0)),
        ],
        out_shape=[
            jax.ShapeDtypeStruct((E, 64), jnp.float32),
            jax.ShapeDtypeStruct((8, 64), jnp.float32),
        ],
    )(gA, gB, ea16, wc16, b1)


def _k2b_body(h1_ref, gd_ref, st1_ref, g1_ref, be1_ref, w2_ref, b2_ref,
              w1e_ref, c2_ref, h2_ref, st_ref):
    i = pl.program_id(0)
    mu = st1_ref[0, :] * (1.0 / E)
    var = st1_ref[1, :] * (1.0 / E) - mu * mu
    sd = jnp.sqrt(var + EPS)
    hn = jnp.maximum((h1_ref[...] - mu[None, :]) / sd[None, :]
                     * g1_ref[0, :][None, :] + be1_ref[0, :][None, :], 0.0)
    e_out = _dotf(hn.astype(bf16), w2_ref[...]) + b2_ref[0, :][None, :]
    h2 = (gd_ref[:, 64:128] + _dotf(e_out.astype(bf16), w1e_ref[...])
          + c2_ref[0, :][None, :])
    h2_ref[...] = h2

    @pl.when(i == 0)
    def _():
        st_ref[...] = jnp.zeros_like(st_ref)

    st_ref[0, :] += jnp.sum(h2, axis=0)
    st_ref[1, :] += jnp.sum(h2 * h2, axis=0)


def _edge_pass_b(h1, gD, st1, g1, be1, w2p16, b2p, w1ep16, c2):
    return pl.pallas_call(
        _k2b_body,
        grid=(EBH,),
        in_specs=[
            pl.BlockSpec((EB, 64), lambda i: (i, 0)),
            pl.BlockSpec((EB, 128), lambda i: (i, 0)),
            pl.BlockSpec((8, 64), lambda i: (0, 0)),
            pl.BlockSpec((1, 64), lambda i: (0, 0)),
            pl.BlockSpec((1, 64), lambda i: (0, 0)),
            pl.BlockSpec((64, 16), lambda i: (0, 0)),
            pl.BlockSpec((1, 16), lambda i: (0, 0)),
            pl.BlockSpec((16, 64), lambda i: (0, 0)),
            pl.BlockSpec((1, 64), lambda i: (0, 0)),
        ],
        out_specs=[
            pl.BlockSpec((EB, 64), lambda i: (i, 0)),
            pl.BlockSpec((8, 64), lambda i: (0, 0)),
        ],
        out_shape=[
            jax.ShapeDtypeStruct((E, 64), jnp.float32),
            jax.ShapeDtypeStruct((8, 64), jnp.float32),
        ],
    )(h1, gD, st1, g1, be1, w2p16, b2p, w1ep16, c2)


def _k2c_body(h2_ref, st2_ref, g2_ref, be2_ref, w2_ref, b2_ref, m_ref):
    mu = st2_ref[0, :] * (1.0 / E)
    var = st2_ref[1, :] * (1.0 / E) - mu * mu
    sd = jnp.sqrt(var + EPS)
    hn = jnp.maximum((h2_ref[...] - mu[None, :]) / sd[None, :]
                     * g2_ref[0, :][None, :] + be2_ref[0, :][None, :], 0.0)
    m_ref[...] = _dotf(hn.astype(bf16), w2_ref[...]) + b2_ref[0, :][None, :]


def _edge_pass_c(h2, st2, g2, be2, w2p16, b2p):
    return pl.pallas_call(
        _k2c_body,
        grid=(EBH,),
        in_specs=[
            pl.BlockSpec((EB, 64), lambda i: (i, 0)),
            pl.BlockSpec((8, 64), lambda i: (0, 0)),
            pl.BlockSpec((1, 64), lambda i: (0, 0)),
            pl.BlockSpec((1, 64), lambda i: (0, 0)),
            pl.BlockSpec((64, 16), lambda i: (0, 0)),
            pl.BlockSpec((1, 16), lambda i: (0, 0)),
        ],
        out_specs=pl.BlockSpec((EB, 16), lambda i: (i, 0)),
        out_shape=jax.ShapeDtypeStruct((E, 16), jnp.float32),
    )(h2, st2, g2, be2, w2p16, b2p)


def _k4_body(acc_ref, batch_ref, rw1_ref, rb1_ref, rg1_ref,
             rbe1_ref, rw2_ref, rb2_ref, out_ref):
    acc = acc_ref[0] + acc_ref[1]
    deg = jnp.maximum(acc[:, 10:11], 1.0)
    node16 = acc / deg
    iota = lax.broadcasted_iota(jnp.int32, (N, G), 1)
    onehot = (batch_ref[...] == iota).astype(jnp.float32)
    cnt = jnp.sum(onehot, axis=0)
    u16 = lax.dot_general(onehot, node16, (((0,), (0,)), ((), ())),
                          preferred_element_type=jnp.float32,
                          precision=lax.Precision.HIGHEST)
    u16 = u16 / jnp.maximum(cnt, 1.0)[:, None]
    h = (_dotf(u16.astype(bf16), rw1_ref[...])
         + rb1_ref[0, :][None, :])
    mu = jnp.mean(h, axis=0)
    var = jnp.mean(h * h, axis=0) - mu * mu
    sd = jnp.sqrt(var + EPS)
    hn = jnp.maximum((h - mu[None, :]) / sd[None, :]
                     * rg1_ref[0, :][None, :] + rbe1_ref[0, :][None, :], 0.0)
    out_ref[...] = (_dotf(hn.astype(bf16), rw2_ref[...])
                    + rb2_ref[0, :][None, :])


def _readout(acc, batch2d, rw1p16, rb1, rg1, rbe1, rw216, rb2):
    return pl.pallas_call(
        _k4_body,
        out_shape=jax.ShapeDtypeStruct((G, N_ACT), jnp.float32),
    )(acc, batch2d, rw1p16, rb1, rg1, rbe1, rw216, rb2)


def kernel(x, edge_index, edge_attr, batch,
           eW1, eb1, eg1, ebeta1, eW2, eb2,
           nW1, nb1, ng1, nbeta1, nW2, nb2,
           rW1, rb1, rg1, rbeta1, rW2, rb2):
    f32 = jnp.float32
    row = edge_index[0].astype(jnp.int32)
    col = edge_index[1].astype(jnp.int32)

    w_all16 = jnp.concatenate(
        [eW1[0:128], nW1[0:128], eW1[128:256],
         jnp.zeros((128, 64), f32)], axis=1).astype(bf16)
    wc16 = eW1[256:272].astype(bf16)
    b1 = eb1.reshape(1, 64)
    g1 = eg1.reshape(1, 64)
    be1 = ebeta1.reshape(1, 64)
    ew2p16 = jnp.zeros((64, 16), f32).at[:, 0:10].set(eW2).astype(bf16)
    eb2p = jnp.zeros((16,), f32).at[0:10].set(eb2).reshape(1, 16)
    nw1ep16 = jnp.zeros((16, 64), f32).at[0:10, :].set(nW1[128:138]).astype(bf16)
    c2 = nb1.reshape(1, 64)
    g2 = ng1.reshape(1, 64)
    be2 = nbeta1.reshape(1, 64)
    nw2p16 = jnp.zeros((64, 16), f32).at[:, 0:10].set(nW2).astype(bf16)
    nb2p = jnp.zeros((16,), f32).at[0:10].set(nb2).at[10].set(1.0)
    nb2p = nb2p.reshape(1, 16)
    rw1p16 = jnp.zeros((16, 64), f32).at[0:10, :].set(rW1).astype(bf16)
    rw216 = rW2.astype(bf16)

    t = _node_tables(x.astype(bf16), w_all16)
    t_ad = t[:, 0:128]
    t_bp = t[:, 128:256]

    gAD, gBP = _sc_gather(t_ad, t_bp, row, col)

    h1, st1 = _edge_pass_a(gAD, gBP, edge_attr.astype(bf16), wc16, b1)
    h2, st2 = _edge_pass_b(h1, gAD, st1, g1, be1, ew2p16, eb2p,
                           nw1ep16, c2)
    m_aug = _edge_pass_c(h2, st2, g2, be2, nw2p16, nb2p)

    acc = _sc_scatter(m_aug, col)

    batch2d = batch.astype(jnp.int32).reshape(N, 1)
    return _readout(acc, batch2d, rw1p16, rb1.reshape(1, 64),
                    rg1.reshape(1, 64), rbeta1.reshape(1, 64),
                    rw216, rb2.reshape(1, N_ACT))


# speedup vs baseline: 1.3176x; 1.0501x over previous
import functools

import jax
import jax.numpy as jnp
from jax import lax
from jax.experimental import pallas as pl
from jax.experimental.pallas import tpu as pltpu
from jax.experimental.pallas import tpu_sc as plsc

N = 10000
E = 320000
D_NODE = 128
G = 64
N_ACT = 8
EPS = 1e-5

EB = 8000
N_EBLK = E // EB

bf16 = jnp.bfloat16

NC = 2
NS = 16
NW = NC * NS
EW = E // NW
GCH = 400
NCH = EW // GCH
GSUB = 80
GNSUB = GCH // GSUB
SCH = 400
SNCH = EW // SCH
SUB = 80
NSUB = SCH // SUB
NZR = 1000
EBH = E // EB


def _dotf(a, b):
    return jnp.dot(a, b, preferred_element_type=jnp.float32)


def _k0_body(x_ref, w_ref, t_ref):
    t_ref[...] = _dotf(x_ref[...], w_ref[...])


def _node_tables(x16, w16):
    return pl.pallas_call(
        _k0_body,
        out_shape=jax.ShapeDtypeStruct((N, 256), jnp.float32),
    )(x16, w16)


def _sc_gather_body(tad, tbp, row, col, gad, gbp,
                    idxr, idxc, bufa, bufb, sema, semb):
    wid = lax.axis_index("s") * NC + lax.axis_index("c")

    def body(i, carry):
        base = wid * EW + i * GCH
        pltpu.sync_copy(row.at[pl.ds(base, GCH)], idxr)
        pltpu.sync_copy(col.at[pl.ds(base, GCH)], idxc)
        cps = []
        for j in range(GNSUB):
            sl = pl.ds(j * GSUB, GSUB)
            cps.append(pltpu.async_copy(tad.at[idxr.at[sl]], bufa.at[sl], sema))
            cps.append(pltpu.async_copy(tbp.at[idxc.at[sl]], bufb.at[sl], semb))
        for cp in cps:
            cp.wait()
        pltpu.sync_copy(bufa, gad.at[pl.ds(base, GCH)])
        pltpu.sync_copy(bufb, gbp.at[pl.ds(base, GCH)])
        return carry

    lax.fori_loop(0, NCH, body, 0)


def _sc_gather(tad, tbp, row, col):
    f32 = jnp.float32
    fn = pl.kernel(
        _sc_gather_body,
        out_type=[
            jax.ShapeDtypeStruct((E, 128), f32),
            jax.ShapeDtypeStruct((E, 128), f32),
        ],
        mesh=plsc.VectorSubcoreMesh(core_axis_name="c", subcore_axis_name="s"),
        scratch_types=[
            pltpu.VMEM((GCH,), jnp.int32),
            pltpu.VMEM((GCH,), jnp.int32),
            pltpu.VMEM((GCH, 128), f32),
            pltpu.VMEM((GCH, 128), f32),
            pltpu.SemaphoreType.DMA,
            pltpu.SemaphoreType.DMA,
        ],
    )
    return fn(tad, tbp, row, col)


def _sc_scatter_body(m2, col1, out, idxw, mbuf, zbuf, acc):
    c = lax.axis_index("c")
    s = lax.axis_index("s")
    wid = s * NC + c

    def zb(i, carry):
        zbuf[i, :] = jnp.zeros((16,), jnp.float32)
        return carry

    lax.fori_loop(0, NZR, zb, 0)

    @pl.when(s < N // NZR)
    def _():
        pltpu.sync_copy(zbuf, acc.at[pl.ds(s * NZR, NZR)])

    plsc.subcore_barrier()

    def body(i, carry):
        base = wid * EW + i * SCH
        pltpu.sync_copy(m2.at[pl.ds(base, SCH)], mbuf)
        for j in range(NSUB):
            pltpu.sync_copy(col1.at[pl.ds(base + j * SUB, SUB)], idxw)
            pltpu.sync_copy(mbuf.at[pl.ds(j * SUB, SUB)],
                            acc.at[idxw], add=True)
        return carry

    lax.fori_loop(0, SNCH, body, 0)
    plsc.subcore_barrier()

    @pl.when(s < N // NZR)
    def _():
        pltpu.sync_copy(acc.at[pl.ds(s * NZR, NZR)],
                        out.at[c].at[pl.ds(s * NZR, NZR)])


def _sc_scatter(m_aug, col):
    f32 = jnp.float32
    fn = pl.kernel(
        _sc_scatter_body,
        out_type=jax.ShapeDtypeStruct((NC, N, 16), f32),
        mesh=plsc.VectorSubcoreMesh(core_axis_name="c", subcore_axis_name="s"),
        scratch_types=[
            pltpu.VMEM((SUB,), jnp.int32),
            pltpu.VMEM((SCH, 16), f32),
            pltpu.VMEM((NZR, 16), f32),
            pltpu.VMEM_SHARED((N, 16), f32),
        ],
        compiler_params=pltpu.CompilerParams(use_tc_tiling_on_sc=False),
    )
    return fn(m_aug, col)


def _k2a_body(ga_ref, gb_ref, ea_ref, wc_ref, b1_ref, h1_ref, st_ref):
    i = pl.program_id(0)
    h1 = (ga_ref[:, 0:64] + gb_ref[:, 0:64]
          + _dotf(ea_ref[...], wc_ref[...])
          + b1_ref[0, :][None, :])
    h1_ref[...] = h1

    @pl.when(i == 0)
    def _():
        st_ref[...] = jnp.zeros_like(st_ref)

    st_ref[0, :] += jnp.sum(h1, axis=0)
    st_ref[1, :] += jnp.sum(h1 * h1, axis=0)


def _edge_pass_a(gA, gB, ea16, wc16, b1):
    return pl.pallas_call(
        _k2a_body,
        grid=(EBH,),
        in_specs=[
            pl.BlockSpec((EB, 128), lambda i: (i, 0)),
            pl.BlockSpec((EB, 128), lambda i: (i, 0)),
            pl.BlockSpec((EB, 16), lambda i: (i, 0)),
            pl.BlockSpec((16, 64), lambda i: (0, 0)),
            pl.BlockSpec((1, 64), lambda i: (0, 0)),
        ],
        out_specs=[
            pl.BlockSpec((EB, 64), lambda i: (i, 0)),
            pl.BlockSpec((8, 64), lambda i: (0, 0)),
        ],
        out_shape=[
            jax.ShapeDtypeStruct((E, 64), jnp.float32),
            jax.ShapeDtypeStruct((8, 64), jnp.float32),
        ],
    )(gA, gB, ea16, wc16, b1)


def _k2b_body(h1_ref, gd_ref, st1_ref, g1_ref, be1_ref, w2_ref, b2_ref,
              w1e_ref, c2_ref, h2_ref, st_ref):
    i = pl.program_id(0)
    mu = st1_ref[0, :] * (1.0 / E)
    var = st1_ref[1, :] * (1.0 / E) - mu * mu
    sd = jnp.sqrt(var + EPS)
    hn = jnp.maximum((h1_ref[...] - mu[None, :]) / sd[None, :]
                     * g1_ref[0, :][None, :] + be1_ref[0, :][None, :], 0.0)
    e_out = _dotf(hn.astype(bf16), w2_ref[...]) + b2_ref[0, :][None, :]
    h2 = (gd_ref[:, 64:128] + _dotf(e_out.astype(bf16), w1e_ref[...])
          + c2_ref[0, :][None, :])
    h2_ref[...] = h2

    @pl.when(i == 0)
    def _():
        st_ref[...] = jnp.zeros_like(st_ref)

    st_ref[0, :] += jnp.sum(h2, axis=0)
    st_ref[1, :] += jnp.sum(h2 * h2, axis=0)


def _edge_pass_b(h1, gD, st1, g1, be1, w2p16, b2p, w1ep16, c2):
    return pl.pallas_call(
        _k2b_body,
        grid=(EBH,),
        in_specs=[
            pl.BlockSpec((EB, 64), lambda i: (i, 0)),
            pl.BlockSpec((EB, 128), lambda i: (i, 0)),
            pl.BlockSpec((8, 64), lambda i: (0, 0)),
            pl.BlockSpec((1, 64), lambda i: (0, 0)),
            pl.BlockSpec((1, 64), lambda i: (0, 0)),
            pl.BlockSpec((64, 16), lambda i: (0, 0)),
            pl.BlockSpec((1, 16), lambda i: (0, 0)),
            pl.BlockSpec((16, 64), lambda i: (0, 0)),
            pl.BlockSpec((1, 64), lambda i: (0, 0)),
        ],
        out_specs=[
            pl.BlockSpec((EB, 64), lambda i: (i, 0)),
            pl.BlockSpec((8, 64), lambda i: (0, 0)),
        ],
        out_shape=[
            jax.ShapeDtypeStruct((E, 64), jnp.float32),
            jax.ShapeDtypeStruct((8, 64), jnp.float32),
        ],
    )(h1, gD, st1, g1, be1, w2p16, b2p, w1ep16, c2)


def _k2c_body(h2_ref, st2_ref, g2_ref, be2_ref, w2_ref, b2_ref, m_ref):
    mu = st2_ref[0, :] * (1.0 / E)
    var = st2_ref[1, :] * (1.0 / E) - mu * mu
    sd = jnp.sqrt(var + EPS)
    hn = jnp.maximum((h2_ref[...] - mu[None, :]) / sd[None, :]
                     * g2_ref[0, :][None, :] + be2_ref[0, :][None, :], 0.0)
    m_ref[...] = _dotf(hn.astype(bf16), w2_ref[...]) + b2_ref[0, :][None, :]


def _edge_pass_c(h2, st2, g2, be2, w2p16, b2p):
    return pl.pallas_call(
        _k2c_body,
        grid=(EBH,),
        in_specs=[
            pl.BlockSpec((EB, 64), lambda i: (i, 0)),
            pl.BlockSpec((8, 64), lambda i: (0, 0)),
            pl.BlockSpec((1, 64), lambda i: (0, 0)),
            pl.BlockSpec((1, 64), lambda i: (0, 0)),
            pl.BlockSpec((64, 16), lambda i: (0, 0)),
            pl.BlockSpec((1, 16), lambda i: (0, 0)),
        ],
        out_specs=pl.BlockSpec((EB, 16), lambda i: (i, 0)),
        out_shape=jax.ShapeDtypeStruct((E, 16), jnp.float32),
    )(h2, st2, g2, be2, w2p16, b2p)


def _k4_body(acc_ref, batch_ref, rw1_ref, rb1_ref, rg1_ref,
             rbe1_ref, rw2_ref, rb2_ref, out_ref):
    acc = acc_ref[0] + acc_ref[1]
    deg = jnp.maximum(acc[:, 10:11], 1.0)
    node16 = acc / deg
    iota = lax.broadcasted_iota(jnp.int32, (N, G), 1)
    onehot = (batch_ref[...] == iota).astype(jnp.float32)
    cnt = jnp.sum(onehot, axis=0)
    u16 = lax.dot_general(onehot, node16, (((0,), (0,)), ((), ())),
                          preferred_element_type=jnp.float32,
                          precision=lax.Precision.HIGHEST)
    u16 = u16 / jnp.maximum(cnt, 1.0)[:, None]
    h = (_dotf(u16.astype(bf16), rw1_ref[...])
         + rb1_ref[0, :][None, :])
    mu = jnp.mean(h, axis=0)
    var = jnp.mean(h * h, axis=0) - mu * mu
    sd = jnp.sqrt(var + EPS)
    hn = jnp.maximum((h - mu[None, :]) / sd[None, :]
                     * rg1_ref[0, :][None, :] + rbe1_ref[0, :][None, :], 0.0)
    out_ref[...] = (_dotf(hn.astype(bf16), rw2_ref[...])
                    + rb2_ref[0, :][None, :])


def _readout(acc, batch2d, rw1p16, rb1, rg1, rbe1, rw216, rb2):
    return pl.pallas_call(
        _k4_body,
        out_shape=jax.ShapeDtypeStruct((G, N_ACT), jnp.float32),
    )(acc, batch2d, rw1p16, rb1, rg1, rbe1, rw216, rb2)


def kernel(x, edge_index, edge_attr, batch,
           eW1, eb1, eg1, ebeta1, eW2, eb2,
           nW1, nb1, ng1, nbeta1, nW2, nb2,
           rW1, rb1, rg1, rbeta1, rW2, rb2):
    f32 = jnp.float32
    row = edge_index[0].astype(jnp.int32)
    col = edge_index[1].astype(jnp.int32)

    w_all16 = jnp.concatenate(
        [eW1[0:128], nW1[0:128], eW1[128:256],
         jnp.zeros((128, 64), f32)], axis=1).astype(bf16)
    wc16 = eW1[256:272].astype(bf16)
    b1 = eb1.reshape(1, 64)
    g1 = eg1.reshape(1, 64)
    be1 = ebeta1.reshape(1, 64)
    ew2p16 = jnp.zeros((64, 16), f32).at[:, 0:10].set(eW2).astype(bf16)
    eb2p = jnp.zeros((16,), f32).at[0:10].set(eb2).reshape(1, 16)
    nw1ep16 = jnp.zeros((16, 64), f32).at[0:10, :].set(nW1[128:138]).astype(bf16)
    c2 = nb1.reshape(1, 64)
    g2 = ng1.reshape(1, 64)
    be2 = nbeta1.reshape(1, 64)
    nw2p16 = jnp.zeros((64, 16), f32).at[:, 0:10].set(nW2).astype(bf16)
    nb2p = jnp.zeros((16,), f32).at[0:10].set(nb2).at[10].set(1.0)
    nb2p = nb2p.reshape(1, 16)
    rw1p16 = jnp.zeros((16, 64), f32).at[0:10, :].set(rW1).astype(bf16)
    rw216 = rW2.astype(bf16)

    t = _node_tables(x.astype(bf16), w_all16)
    t_ad = t[:, 0:128]
    t_bp = t[:, 128:256]

    gAD, gBP = _sc_gather(t_ad, t_bp, row, col)

    h1, st1 = _edge_pass_a(gAD, gBP, edge_attr.astype(bf16), wc16, b1)
    h2, st2 = _edge_pass_b(h1, gAD, st1, g1, be1, ew2p16, eb2p,
                           nw1ep16, c2)
    m_aug = _edge_pass_c(h2, st2, g2, be2, nw2p16, nb2p)

    acc = _sc_scatter(m_aug, col)

    batch2d = batch.astype(jnp.int32).reshape(N, 1)
    return _readout(acc, batch2d, rw1p16, rb1.reshape(1, 64),
                    rg1.reshape(1, 64), rbeta1.reshape(1, 64),
                    rw216, rb2.reshape(1, N_ACT))


# speedup vs baseline: 1.3252x; 1.0058x over previous
import functools

import jax
import jax.numpy as jnp
from jax import lax
from jax.experimental import pallas as pl
from jax.experimental.pallas import tpu as pltpu
from jax.experimental.pallas import tpu_sc as plsc

N = 10000
E = 320000
D_NODE = 128
G = 64
N_ACT = 8
EPS = 1e-5

EB = 10000
N_EBLK = E // EB

bf16 = jnp.bfloat16

NC = 2
NS = 16
NW = NC * NS
EW = E // NW
GCH = 400
NCH = EW // GCH
GSUB = 80
GNSUB = GCH // GSUB
SCH = 400
SNCH = EW // SCH
SUB = 80
NSUB = SCH // SUB
NZR = 1000
EBH = E // EB


def _dotf(a, b):
    return jnp.dot(a, b, preferred_element_type=jnp.float32)


def _k0_body(x_ref, w_ref, t_ref):
    t_ref[...] = _dotf(x_ref[...], w_ref[...])


def _node_tables(x16, w16):
    return pl.pallas_call(
        _k0_body,
        out_shape=jax.ShapeDtypeStruct((N, 256), jnp.float32),
    )(x16, w16)


def _sc_gather_body(tad, tbp, row, col, gad, gbp,
                    idxr, idxc, bufa, bufb, sema, semb):
    wid = lax.axis_index("s") * NC + lax.axis_index("c")

    def body(i, carry):
        base = wid * EW + i * GCH
        pltpu.sync_copy(row.at[pl.ds(base, GCH)], idxr)
        pltpu.sync_copy(col.at[pl.ds(base, GCH)], idxc)
        cps = []
        for j in range(GNSUB):
            sl = pl.ds(j * GSUB, GSUB)
            cps.append(pltpu.async_copy(tad.at[idxr.at[sl]], bufa.at[sl], sema))
            cps.append(pltpu.async_copy(tbp.at[idxc.at[sl]], bufb.at[sl], semb))
        for cp in cps:
            cp.wait()
        pltpu.sync_copy(bufa, gad.at[pl.ds(base, GCH)])
        pltpu.sync_copy(bufb, gbp.at[pl.ds(base, GCH)])
        return carry

    lax.fori_loop(0, NCH, body, 0)


def _sc_gather(tad, tbp, row, col):
    f32 = jnp.float32
    fn = pl.kernel(
        _sc_gather_body,
        out_type=[
            jax.ShapeDtypeStruct((E, 128), f32),
            jax.ShapeDtypeStruct((E, 128), f32),
        ],
        mesh=plsc.VectorSubcoreMesh(core_axis_name="c", subcore_axis_name="s"),
        scratch_types=[
            pltpu.VMEM((GCH,), jnp.int32),
            pltpu.VMEM((GCH,), jnp.int32),
            pltpu.VMEM((GCH, 128), f32),
            pltpu.VMEM((GCH, 128), f32),
            pltpu.SemaphoreType.DMA,
            pltpu.SemaphoreType.DMA,
        ],
    )
    return fn(tad, tbp, row, col)


def _sc_scatter_body(m2, col1, out, idxw, mbuf, zbuf, acc):
    c = lax.axis_index("c")
    s = lax.axis_index("s")
    wid = s * NC + c

    def zb(i, carry):
        zbuf[i, :] = jnp.zeros((16,), jnp.float32)
        return carry

    lax.fori_loop(0, NZR, zb, 0)

    @pl.when(s < N // NZR)
    def _():
        pltpu.sync_copy(zbuf, acc.at[pl.ds(s * NZR, NZR)])

    plsc.subcore_barrier()

    def body(i, carry):
        base = wid * EW + i * SCH
        pltpu.sync_copy(m2.at[pl.ds(base, SCH)], mbuf)
        for j in range(NSUB):
            pltpu.sync_copy(col1.at[pl.ds(base + j * SUB, SUB)], idxw)
            pltpu.sync_copy(mbuf.at[pl.ds(j * SUB, SUB)],
                            acc.at[idxw], add=True)
        return carry

    lax.fori_loop(0, SNCH, body, 0)
    plsc.subcore_barrier()

    @pl.when(s < N // NZR)
    def _():
        pltpu.sync_copy(acc.at[pl.ds(s * NZR, NZR)],
                        out.at[c].at[pl.ds(s * NZR, NZR)])


def _sc_scatter(m_aug, col):
    f32 = jnp.float32
    fn = pl.kernel(
        _sc_scatter_body,
        out_type=jax.ShapeDtypeStruct((NC, N, 16), f32),
        mesh=plsc.VectorSubcoreMesh(core_axis_name="c", subcore_axis_name="s"),
        scratch_types=[
            pltpu.VMEM((SUB,), jnp.int32),
            pltpu.VMEM((SCH, 16), f32),
            pltpu.VMEM((NZR, 16), f32),
            pltpu.VMEM_SHARED((N, 16), f32),
        ],
        compiler_params=pltpu.CompilerParams(use_tc_tiling_on_sc=False),
    )
    return fn(m_aug, col)


def _k2a_body(ga_ref, gb_ref, ea_ref, wc_ref, b1_ref, h1_ref, st_ref):
    i = pl.program_id(0)
    h1 = (ga_ref[:, 0:64] + gb_ref[:, 0:64]
          + _dotf(ea_ref[...], wc_ref[...])
          + b1_ref[0, :][None, :])
    h1_ref[...] = h1

    @pl.when(i == 0)
    def _():
        st_ref[...] = jnp.zeros_like(st_ref)

    st_ref[0, :] += jnp.sum(h1, axis=0)
    st_ref[1, :] += jnp.sum(h1 * h1, axis=0)


def _edge_pass_a(gA, gB, ea16, wc16, b1):
    return pl.pallas_call(
        _k2a_body,
        grid=(EBH,),
        in_specs=[
            pl.BlockSpec((EB, 128), lambda i: (i, 0)),
            pl.BlockSpec((EB, 128), lambda i: (i, 0)),
            pl.BlockSpec((EB, 16), lambda i: (i, 0)),
            pl.BlockSpec((16, 64), lambda i: (0, 0)),
            pl.BlockSpec((1, 64), lambda i: (0, 0)),
        ],
        out_specs=[
            pl.BlockSpec((EB, 64), lambda i: (i, 0)),
            pl.BlockSpec((8, 64), lambda i: (0, 0)),
        ],
        out_shape=[
            jax.ShapeDtypeStruct((E, 64), jnp.float32),
            jax.ShapeDtypeStruct((8, 64), jnp.float32),
        ],
    )(gA, gB, ea16, wc16, b1)


def _k2b_body(h1_ref, gd_ref, st1_ref, g1_ref, be1_ref, w2_ref, b2_ref,
              w1e_ref, c2_ref, h2_ref, st_ref):
    i = pl.program_id(0)
    mu = st1_ref[0, :] * (1.0 / E)
    var = st1_ref[1, :] * (1.0 / E) - mu * mu
    sd = jnp.sqrt(var + EPS)
    hn = jnp.maximum((h1_ref[...] - mu[None, :]) / sd[None, :]
                     * g1_ref[0, :][None, :] + be1_ref[0, :][None, :], 0.0)
    e_out = _dotf(hn.astype(bf16), w2_ref[...]) + b2_ref[0, :][None, :]
    h2 = (gd_ref[:, 64:128] + _dotf(e_out.astype(bf16), w1e_ref[...])
          + c2_ref[0, :][None, :])
    h2_ref[...] = h2

    @pl.when(i == 0)
    def _():
        st_ref[...] = jnp.zeros_like(st_ref)

    st_ref[0, :] += jnp.sum(h2, axis=0)
    st_ref[1, :] += jnp.sum(h2 * h2, axis=0)


def _edge_pass_b(h1, gD, st1, g1, be1, w2p16, b2p, w1ep16, c2):
    return pl.pallas_call(
        _k2b_body,
        grid=(EBH,),
        in_specs=[
            pl.BlockSpec((EB, 64), lambda i: (i, 0)),
            pl.BlockSpec((EB, 128), lambda i: (i, 0)),
            pl.BlockSpec((8, 64), lambda i: (0, 0)),
            pl.BlockSpec((1, 64), lambda i: (0, 0)),
            pl.BlockSpec((1, 64), lambda i: (0, 0)),
            pl.BlockSpec((64, 16), lambda i: (0, 0)),
            pl.BlockSpec((1, 16), lambda i: (0, 0)),
            pl.BlockSpec((16, 64), lambda i: (0, 0)),
            pl.BlockSpec((1, 64), lambda i: (0, 0)),
        ],
        out_specs=[
            pl.BlockSpec((EB, 64), lambda i: (i, 0)),
            pl.BlockSpec((8, 64), lambda i: (0, 0)),
        ],
        out_shape=[
            jax.ShapeDtypeStruct((E, 64), jnp.float32),
            jax.ShapeDtypeStruct((8, 64), jnp.float32),
        ],
    )(h1, gD, st1, g1, be1, w2p16, b2p, w1ep16, c2)


def _k2c_body(h2_ref, st2_ref, g2_ref, be2_ref, w2_ref, b2_ref, m_ref):
    mu = st2_ref[0, :] * (1.0 / E)
    var = st2_ref[1, :] * (1.0 / E) - mu * mu
    sd = jnp.sqrt(var + EPS)
    hn = jnp.maximum((h2_ref[...] - mu[None, :]) / sd[None, :]
                     * g2_ref[0, :][None, :] + be2_ref[0, :][None, :], 0.0)
    m_ref[...] = _dotf(hn.astype(bf16), w2_ref[...]) + b2_ref[0, :][None, :]


def _edge_pass_c(h2, st2, g2, be2, w2p16, b2p):
    return pl.pallas_call(
        _k2c_body,
        grid=(EBH,),
        in_specs=[
            pl.BlockSpec((EB, 64), lambda i: (i, 0)),
            pl.BlockSpec((8, 64), lambda i: (0, 0)),
            pl.BlockSpec((1, 64), lambda i: (0, 0)),
            pl.BlockSpec((1, 64), lambda i: (0, 0)),
            pl.BlockSpec((64, 16), lambda i: (0, 0)),
            pl.BlockSpec((1, 16), lambda i: (0, 0)),
        ],
        out_specs=pl.BlockSpec((EB, 16), lambda i: (i, 0)),
        out_shape=jax.ShapeDtypeStruct((E, 16), jnp.float32),
    )(h2, st2, g2, be2, w2p16, b2p)


def _k4_body(acc_ref, batch_ref, rw1_ref, rb1_ref, rg1_ref,
             rbe1_ref, rw2_ref, rb2_ref, out_ref):
    acc = acc_ref[0] + acc_ref[1]
    deg = jnp.maximum(acc[:, 10:11], 1.0)
    node16 = acc / deg
    iota = lax.broadcasted_iota(jnp.int32, (N, G), 1)
    onehot = (batch_ref[...] == iota).astype(jnp.float32)
    cnt = jnp.sum(onehot, axis=0)
    u16 = lax.dot_general(onehot, node16, (((0,), (0,)), ((), ())),
                          preferred_element_type=jnp.float32,
                          precision=lax.Precision.HIGHEST)
    u16 = u16 / jnp.maximum(cnt, 1.0)[:, None]
    h = (_dotf(u16.astype(bf16), rw1_ref[...])
         + rb1_ref[0, :][None, :])
    mu = jnp.mean(h, axis=0)
    var = jnp.mean(h * h, axis=0) - mu * mu
    sd = jnp.sqrt(var + EPS)
    hn = jnp.maximum((h - mu[None, :]) / sd[None, :]
                     * rg1_ref[0, :][None, :] + rbe1_ref[0, :][None, :], 0.0)
    out_ref[...] = (_dotf(hn.astype(bf16), rw2_ref[...])
                    + rb2_ref[0, :][None, :])


def _readout(acc, batch2d, rw1p16, rb1, rg1, rbe1, rw216, rb2):
    return pl.pallas_call(
        _k4_body,
        out_shape=jax.ShapeDtypeStruct((G, N_ACT), jnp.float32),
    )(acc, batch2d, rw1p16, rb1, rg1, rbe1, rw216, rb2)


def kernel(x, edge_index, edge_attr, batch,
           eW1, eb1, eg1, ebeta1, eW2, eb2,
           nW1, nb1, ng1, nbeta1, nW2, nb2,
           rW1, rb1, rg1, rbeta1, rW2, rb2):
    f32 = jnp.float32
    row = edge_index[0].astype(jnp.int32)
    col = edge_index[1].astype(jnp.int32)

    w_all16 = jnp.concatenate(
        [eW1[0:128], nW1[0:128], eW1[128:256],
         jnp.zeros((128, 64), f32)], axis=1).astype(bf16)
    wc16 = eW1[256:272].astype(bf16)
    b1 = eb1.reshape(1, 64)
    g1 = eg1.reshape(1, 64)
    be1 = ebeta1.reshape(1, 64)
    ew2p16 = jnp.zeros((64, 16), f32).at[:, 0:10].set(eW2).astype(bf16)
    eb2p = jnp.zeros((16,), f32).at[0:10].set(eb2).reshape(1, 16)
    nw1ep16 = jnp.zeros((16, 64), f32).at[0:10, :].set(nW1[128:138]).astype(bf16)
    c2 = nb1.reshape(1, 64)
    g2 = ng1.reshape(1, 64)
    be2 = nbeta1.reshape(1, 64)
    nw2p16 = jnp.zeros((64, 16), f32).at[:, 0:10].set(nW2).astype(bf16)
    nb2p = jnp.zeros((16,), f32).at[0:10].set(nb2).at[10].set(1.0)
    nb2p = nb2p.reshape(1, 16)
    rw1p16 = jnp.zeros((16, 64), f32).at[0:10, :].set(rW1).astype(bf16)
    rw216 = rW2.astype(bf16)

    t = _node_tables(x.astype(bf16), w_all16)
    t_ad = t[:, 0:128]
    t_bp = t[:, 128:256]

    gAD, gBP = _sc_gather(t_ad, t_bp, row, col)

    h1, st1 = _edge_pass_a(gAD, gBP, edge_attr.astype(bf16), wc16, b1)
    h2, st2 = _edge_pass_b(h1, gAD, st1, g1, be1, ew2p16, eb2p,
                           nw1ep16, c2)
    m_aug = _edge_pass_c(h2, st2, g2, be2, nw2p16, nb2p)

    acc = _sc_scatter(m_aug, col)

    batch2d = batch.astype(jnp.int32).reshape(N, 1)
    return _readout(acc, batch2d, rw1p16, rb1.reshape(1, 64),
                    rg1.reshape(1, 64), rbeta1.reshape(1, 64),
                    rw216, rb2.reshape(1, N_ACT))
